# baseline (speedup 1.0000x reference)
import jax
import jax.numpy as jnp
from jax import lax
from jax.experimental import pallas as pl
from jax.experimental.pallas import tpu as pltpu
from jax.experimental.pallas import tpu_sc as plsc

N = 10000
D = 128
H = 128
G = 64
E = 320000

NC = 2
NS = 16
NW = NC * NS
K = 128
CPW = 80
CPW0 = 40
CPW1 = 120
E_PAD = NW * K * CPW
NBUF = 2
N_ACC = 10240
RPT = N_ACC // NS
WD = 16

RB = 2000
NBLK = N // RB


def _copy_idx(flat_ref, off, dst_ref):
    for m in range(K // 16):
        dst_ref[pl.ds(m * 16, 16)] = flat_ref[pl.ds(off + m * 16, 16)]


def _sc_deg_body(dst_hbm, out_hbm, idx_f, d0, d1, ones_v, zb, acc, s0, s1):
    c = lax.axis_index("c")
    s = lax.axis_index("s")
    wid = c * NS + s
    idx_d = (d0, d1)
    ssem = (s0, s1)

    @pl.loop(0, K)
    def _fill(i):
        zb[i, :] = jnp.zeros((WD,), jnp.float32)
        ones_v[i, :] = jnp.ones((WD,), jnp.float32)

    pltpu.sync_copy(dst_hbm.at[pl.ds(wid * (CPW * K), CPW * K)], idx_f)

    @pl.loop(0, RPT // K)
    def _zero(j):
        pltpu.sync_copy(zb, acc.at[pl.ds(s * RPT + j * K, K)])

    plsc.subcore_barrier()

    @pl.loop(0, CPW, step=2)
    def _edges(q):
        for b in range(2):
            j = q + b

            @pl.when(j >= 2)
            def _prev():
                pltpu.make_async_copy(ones_v, acc.at[idx_d[b]],
                                      ssem[b]).wait()

            _copy_idx(idx_f, j * K, idx_d[b])
            pltpu.async_copy(ones_v, acc.at[idx_d[b]], ssem[b], add=True)

    for b in range(2):
        pltpu.make_async_copy(ones_v, acc.at[idx_d[b]], ssem[b]).wait()

    plsc.subcore_barrier()
    pltpu.sync_copy(acc.at[pl.ds(s * RPT, RPT)],
                    out_hbm.at[c, pl.ds(s * RPT, RPT)])


_sc_deg = pl.kernel(
    _sc_deg_body,
    out_type=jax.ShapeDtypeStruct((NC, N_ACC, WD), jnp.float32),
    mesh=plsc.VectorSubcoreMesh(core_axis_name="c", subcore_axis_name="s"),
    scratch_types=[
        pltpu.VMEM((CPW * K,), jnp.int32),
        pltpu.VMEM((K,), jnp.int32),
        pltpu.VMEM((K,), jnp.int32),
        pltpu.VMEM((K, WD), jnp.float32),
        pltpu.VMEM((K, WD), jnp.float32),
        pltpu.VMEM_SHARED((N_ACC, WD), jnp.float32),
        pltpu.SemaphoreType.DMA,
        pltpu.SemaphoreType.DMA,
    ],
)


def _sc_agg_body(g_hbm, src_hbm, dst_hbm, out_hbm, idx_sb, idx_db,
                 d0, d1, rows0, rows1, acc, gs0, gs1, ss0, ss1):
    c = lax.axis_index("c")
    s = lax.axis_index("s")
    idx_d = (d0, d1)
    rows = (rows0, rows1)
    gsem = (gs0, gs1)
    ssem = (ss0, ss1)
    cpw = jnp.where(c == 0, CPW0, CPW1)
    base = jnp.where(c == 0, s * CPW0, NS * CPW0 + s * CPW1) * K

    @pl.loop(0, K)
    def _fill(i):
        for j in range(H // 16):
            rows0[i, pl.ds(j * 16, 16)] = jnp.zeros((16,), jnp.float32)

    @pl.loop(0, RPT // K)
    def _zero(j):
        pltpu.sync_copy(rows0, acc.at[pl.ds(s * RPT + j * K, K)])

    pltpu.sync_copy(src_hbm.at[pl.ds(base, NBUF * K)], idx_sb)
    pltpu.sync_copy(dst_hbm.at[pl.ds(base, NBUF * K)], idx_db)
    for b in range(NBUF):
        pltpu.async_copy(g_hbm.at[idx_sb.at[pl.ds(b * K, K)]], rows[b],
                         gsem[b])

    plsc.subcore_barrier()

    @pl.loop(0, cpw, step=NBUF)
    def _edges(q):
        for b in range(NBUF):
            pltpu.make_async_copy(g_hbm.at[idx_sb.at[pl.ds(b * K, K)]],
                                  rows[b], gsem[b]).wait()
            _copy_idx(idx_db, b * K, idx_d[b])
            pltpu.async_copy(rows[b], acc.at[idx_d[b]], ssem[b], add=True)
        nbase = base + lax.rem(q + NBUF, cpw) * K
        pltpu.sync_copy(src_hbm.at[pl.ds(nbase, NBUF * K)], idx_sb)
        pltpu.sync_copy(dst_hbm.at[pl.ds(nbase, NBUF * K)], idx_db)
        for b in range(NBUF):
            pltpu.make_async_copy(rows[b], acc.at[idx_d[b]],
                                  ssem[b]).wait()
            pltpu.async_copy(g_hbm.at[idx_sb.at[pl.ds(b * K, K)]],
                             rows[b], gsem[b])

    for b in range(NBUF):
        pltpu.make_async_copy(g_hbm.at[idx_sb.at[pl.ds(b * K, K)]],
                              rows[b], gsem[b]).wait()

    plsc.subcore_barrier()
    pltpu.sync_copy(acc.at[pl.ds(s * RPT, RPT)],
                    out_hbm.at[c, pl.ds(s * RPT, RPT)])


_sc_agg = pl.kernel(
    _sc_agg_body,
    out_type=jax.ShapeDtypeStruct((NC, N_ACC, H), jnp.float32),
    mesh=plsc.VectorSubcoreMesh(core_axis_name="c", subcore_axis_name="s"),
    scratch_types=(
        [pltpu.VMEM((NBUF * K,), jnp.int32)] * 2
        + [pltpu.VMEM((K,), jnp.int32)] * NBUF
        + [pltpu.VMEM((K, H), jnp.float32)] * NBUF
        + [pltpu.VMEM_SHARED((N_ACC, H), jnp.float32)]
        + [pltpu.SemaphoreType.DMA] * (2 * NBUF)
    ),
)


def _tc_scale_body(x_ref, w1_ref, degp_ref, g1_ref, dinv_ref):
    deg = degp_ref[0, :, 0:1] + degp_ref[1, :, 0:1] + 1.0
    dinv = lax.rsqrt(deg)
    h1 = jnp.dot(x_ref[...], w1_ref[...], preferred_element_type=jnp.float32)
    g1_ref[...] = h1 * dinv
    dinv_ref[...] = dinv


_tc_scale = pl.pallas_call(
    _tc_scale_body,
    grid=(NBLK,),
    in_specs=[
        pl.BlockSpec((RB, D), lambda i: (i, 0)),
        pl.BlockSpec((D, H), lambda i: (0, 0)),
        pl.BlockSpec((NC, RB, WD), lambda i: (0, i, 0)),
    ],
    out_specs=[
        pl.BlockSpec((RB, H), lambda i: (i, 0)),
        pl.BlockSpec((RB, 1), lambda i: (i, 0)),
    ],
    out_shape=[
        jax.ShapeDtypeStruct((N, H), jnp.float32),
        jax.ShapeDtypeStruct((N, 1), jnp.float32),
    ],
)


def _tc_mid_body(aggp_ref, g_ref, dinv_ref, b_ref, w2_ref, gn_ref, out_ref):
    agg = aggp_ref[0] + aggp_ref[1] + g_ref[...]
    dinv = dinv_ref[...]
    outl = jnp.maximum(agg * dinv + b_ref[...], 0.0)
    out_ref[...] = outl
    h2 = jnp.dot(outl, w2_ref[...], preferred_element_type=jnp.float32)
    gn_ref[...] = h2 * dinv


_tc_mid = pl.pallas_call(
    _tc_mid_body,
    grid=(NBLK,),
    in_specs=[
        pl.BlockSpec((NC, RB, H), lambda i: (0, i, 0)),
        pl.BlockSpec((RB, H), lambda i: (i, 0)),
        pl.BlockSpec((RB, 1), lambda i: (i, 0)),
        pl.BlockSpec((1, H), lambda i: (0, 0)),
        pl.BlockSpec((H, H), lambda i: (0, 0)),
    ],
    out_specs=[
        pl.BlockSpec((RB, H), lambda i: (i, 0)),
        pl.BlockSpec((RB, H), lambda i: (i, 0)),
    ],
    out_shape=[
        jax.ShapeDtypeStruct((N, H), jnp.float32),
        jax.ShapeDtypeStruct((N, H), jnp.float32),
    ],
    input_output_aliases={1: 0},
)


def _tc_pool_body(out2_ref, batch_ref, wfc_ref, bfc_ref, out_ref,
                  s_acc, c_acc):
    i = pl.program_id(0)
    out2 = out2_ref[...]
    seg = (batch_ref[...] == lax.broadcasted_iota(jnp.int32, (RB, G), 1))
    seg = seg.astype(jnp.float32)
    part_s = lax.dot_general(seg, out2, (((0,), (0,)), ((), ())),
                             preferred_element_type=jnp.float32)
    ones = jnp.ones((RB, 1), jnp.float32)
    part_c = lax.dot_general(seg, ones, (((0,), (0,)), ((), ())),
                             preferred_element_type=jnp.float32)

    @pl.when(i == 0)
    def _():
        s_acc[...] = part_s
        c_acc[...] = part_c

    @pl.when(i > 0)
    def _():
        s_acc[...] += part_s
        c_acc[...] += part_c

    @pl.when(i == NBLK - 1)
    def _():
        pooled = s_acc[...] / jnp.maximum(c_acc[...], 1.0)
        out_ref[...] = (jnp.dot(pooled, wfc_ref[...],
                                preferred_element_type=jnp.float32)
                        + bfc_ref[...])


_tc_pool = pl.pallas_call(
    _tc_pool_body,
    grid=(NBLK,),
    in_specs=[
        pl.BlockSpec((RB, H), lambda i: (i, 0)),
        pl.BlockSpec((RB, 1), lambda i: (i, 0)),
        pl.BlockSpec((H, 1), lambda i: (0, 0)),
        pl.BlockSpec((1, 1), lambda i: (0, 0)),
    ],
    out_specs=pl.BlockSpec((G, 1), lambda i: (0, 0)),
    out_shape=jax.ShapeDtypeStruct((G, 1), jnp.float32),
    scratch_shapes=[
        pltpu.VMEM((G, H), jnp.float32),
        pltpu.VMEM((G, 1), jnp.float32),
    ],
)


def kernel(x, edge_index, batch, W1, b1, W2, b2, Wfc, bfc):
    pad = E_PAD - E
    srcp = jnp.concatenate([edge_index[0], jnp.zeros((pad,), jnp.int32)])
    dstp = jnp.concatenate([edge_index[1], jnp.full((pad,), N, jnp.int32)])

    degp = _sc_deg(dstp)
    g1, dinv = _tc_scale(x, W1, degp)

    agg1 = _sc_agg(g1, srcp, dstp)
    g2, _ = _tc_mid(agg1, g1, dinv, b1.reshape(1, H), W2)
    agg2 = _sc_agg(g2, srcp, dstp)
    _, out2 = _tc_mid(agg2, g2, dinv, b2.reshape(1, H), W2)

    out = _tc_pool(out2, batch.reshape(N, 1), Wfc, bfc.reshape(1, 1))
    return out.reshape(G)


# speedup vs baseline: 1.2173x; 1.2173x over previous
import jax
import jax.numpy as jnp
from jax import lax
from jax.experimental import pallas as pl
from jax.experimental.pallas import tpu as pltpu
from jax.experimental.pallas import tpu_sc as plsc

N = 10000
D = 128
H = 128
G = 64
E = 320000

NC = 2
NS = 16
NW = NC * NS
K = 128
CPW = 80
CPW0 = 120
CPW1 = 40
E_PAD = NW * K * CPW
NBUF = 2
N_ACC = 10240
RPT = N_ACC // NS
WD = 16

RB = 2000
NBLK = N // RB


def _copy_idx(flat_ref, off, dst_ref):
    for m in range(K // 16):
        dst_ref[pl.ds(m * 16, 16)] = flat_ref[pl.ds(off + m * 16, 16)]


def _sc_deg_body(dst_hbm, out_hbm, idx_f, d0, d1, ones_v, zb, acc, s0, s1):
    c = lax.axis_index("c")
    s = lax.axis_index("s")
    wid = c * NS + s
    idx_d = (d0, d1)
    ssem = (s0, s1)

    @pl.loop(0, K)
    def _fill(i):
        zb[i, :] = jnp.zeros((WD,), jnp.float32)
        ones_v[i, :] = jnp.ones((WD,), jnp.float32)

    pltpu.sync_copy(dst_hbm.at[pl.ds(wid * (CPW * K), CPW * K)], idx_f)

    @pl.loop(0, RPT // K)
    def _zero(j):
        pltpu.sync_copy(zb, acc.at[pl.ds(s * RPT + j * K, K)])

    plsc.subcore_barrier()

    @pl.loop(0, CPW, step=2)
    def _edges(q):
        for b in range(2):
            j = q + b

            @pl.when(j >= 2)
            def _prev():
                pltpu.make_async_copy(ones_v, acc.at[idx_d[b]],
                                      ssem[b]).wait()

            _copy_idx(idx_f, j * K, idx_d[b])
            pltpu.async_copy(ones_v, acc.at[idx_d[b]], ssem[b], add=True)

    for b in range(2):
        pltpu.make_async_copy(ones_v, acc.at[idx_d[b]], ssem[b]).wait()

    plsc.subcore_barrier()
    pltpu.sync_copy(acc.at[pl.ds(s * RPT, RPT)],
                    out_hbm.at[c, pl.ds(s * RPT, RPT)])


_sc_deg = pl.kernel(
    _sc_deg_body,
    out_type=jax.ShapeDtypeStruct((NC, N_ACC, WD), jnp.float32),
    mesh=plsc.VectorSubcoreMesh(core_axis_name="c", subcore_axis_name="s"),
    scratch_types=[
        pltpu.VMEM((CPW * K,), jnp.int32),
        pltpu.VMEM((K,), jnp.int32),
        pltpu.VMEM((K,), jnp.int32),
        pltpu.VMEM((K, WD), jnp.float32),
        pltpu.VMEM((K, WD), jnp.float32),
        pltpu.VMEM_SHARED((N_ACC, WD), jnp.float32),
        pltpu.SemaphoreType.DMA,
        pltpu.SemaphoreType.DMA,
    ],
)


def _sc_agg_body(g_hbm, src_hbm, dst_hbm, out_hbm, idx_sb, idx_db,
                 d0, d1, rows0, rows1, acc, gs0, gs1, ss0, ss1):
    c = lax.axis_index("c")
    s = lax.axis_index("s")
    idx_d = (d0, d1)
    rows = (rows0, rows1)
    gsem = (gs0, gs1)
    ssem = (ss0, ss1)
    cpw = jnp.where(c == 0, CPW0, CPW1)
    base = jnp.where(c == 0, s * CPW0, NS * CPW0 + s * CPW1) * K

    @pl.loop(0, K)
    def _fill(i):
        for j in range(H // 16):
            rows0[i, pl.ds(j * 16, 16)] = jnp.zeros((16,), jnp.float32)

    @pl.loop(0, RPT // K)
    def _zero(j):
        pltpu.sync_copy(rows0, acc.at[pl.ds(s * RPT + j * K, K)])

    pltpu.sync_copy(src_hbm.at[pl.ds(base, NBUF * K)], idx_sb)
    pltpu.sync_copy(dst_hbm.at[pl.ds(base, NBUF * K)], idx_db)
    for b in range(NBUF):
        pltpu.async_copy(g_hbm.at[idx_sb.at[pl.ds(b * K, K)]], rows[b],
                         gsem[b])

    plsc.subcore_barrier()

    @pl.loop(0, cpw, step=NBUF)
    def _edges(q):
        for b in range(NBUF):
            pltpu.make_async_copy(g_hbm.at[idx_sb.at[pl.ds(b * K, K)]],
                                  rows[b], gsem[b]).wait()
            _copy_idx(idx_db, b * K, idx_d[b])
            pltpu.async_copy(rows[b], acc.at[idx_d[b]], ssem[b], add=True)
        nbase = base + lax.rem(q + NBUF, cpw) * K
        pltpu.sync_copy(src_hbm.at[pl.ds(nbase, NBUF * K)], idx_sb)
        pltpu.sync_copy(dst_hbm.at[pl.ds(nbase, NBUF * K)], idx_db)
        for b in range(NBUF):
            pltpu.make_async_copy(rows[b], acc.at[idx_d[b]],
                                  ssem[b]).wait()
            pltpu.async_copy(g_hbm.at[idx_sb.at[pl.ds(b * K, K)]],
                             rows[b], gsem[b])

    for b in range(NBUF):
        pltpu.make_async_copy(g_hbm.at[idx_sb.at[pl.ds(b * K, K)]],
                              rows[b], gsem[b]).wait()

    plsc.subcore_barrier()
    pltpu.sync_copy(acc.at[pl.ds(s * RPT, RPT)],
                    out_hbm.at[c, pl.ds(s * RPT, RPT)])


_sc_agg = pl.kernel(
    _sc_agg_body,
    out_type=jax.ShapeDtypeStruct((NC, N_ACC, H), jnp.float32),
    mesh=plsc.VectorSubcoreMesh(core_axis_name="c", subcore_axis_name="s"),
    scratch_types=(
        [pltpu.VMEM((NBUF * K,), jnp.int32)] * 2
        + [pltpu.VMEM((K,), jnp.int32)] * NBUF
        + [pltpu.VMEM((K, H), jnp.float32)] * NBUF
        + [pltpu.VMEM_SHARED((N_ACC, H), jnp.float32)]
        + [pltpu.SemaphoreType.DMA] * (2 * NBUF)
    ),
)


def _tc_scale_body(x_ref, w1_ref, degp_ref, g1_ref, dinv_ref):
    deg = degp_ref[0, :, 0:1] + degp_ref[1, :, 0:1] + 1.0
    dinv = lax.rsqrt(deg)
    h1 = jnp.dot(x_ref[...], w1_ref[...], preferred_element_type=jnp.float32)
    g1_ref[...] = h1 * dinv
    dinv_ref[...] = dinv


_tc_scale = pl.pallas_call(
    _tc_scale_body,
    grid=(NBLK,),
    in_specs=[
        pl.BlockSpec((RB, D), lambda i: (i, 0)),
        pl.BlockSpec((D, H), lambda i: (0, 0)),
        pl.BlockSpec((NC, RB, WD), lambda i: (0, i, 0)),
    ],
    out_specs=[
        pl.BlockSpec((RB, H), lambda i: (i, 0)),
        pl.BlockSpec((RB, 1), lambda i: (i, 0)),
    ],
    out_shape=[
        jax.ShapeDtypeStruct((N, H), jnp.float32),
        jax.ShapeDtypeStruct((N, 1), jnp.float32),
    ],
)


def _tc_mid_body(aggp_ref, g_ref, dinv_ref, b_ref, w2_ref, gn_ref, out_ref):
    agg = aggp_ref[0] + aggp_ref[1] + g_ref[...]
    dinv = dinv_ref[...]
    outl = jnp.maximum(agg * dinv + b_ref[...], 0.0)
    out_ref[...] = outl
    h2 = jnp.dot(outl, w2_ref[...], preferred_element_type=jnp.float32)
    gn_ref[...] = h2 * dinv


_tc_mid = pl.pallas_call(
    _tc_mid_body,
    grid=(NBLK,),
    in_specs=[
        pl.BlockSpec((NC, RB, H), lambda i: (0, i, 0)),
        pl.BlockSpec((RB, H), lambda i: (i, 0)),
        pl.BlockSpec((RB, 1), lambda i: (i, 0)),
        pl.BlockSpec((1, H), lambda i: (0, 0)),
        pl.BlockSpec((H, H), lambda i: (0, 0)),
    ],
    out_specs=[
        pl.BlockSpec((RB, H), lambda i: (i, 0)),
        pl.BlockSpec((RB, H), lambda i: (i, 0)),
    ],
    out_shape=[
        jax.ShapeDtypeStruct((N, H), jnp.float32),
        jax.ShapeDtypeStruct((N, H), jnp.float32),
    ],
    input_output_aliases={1: 0},
)


def _tc_pool_body(out2_ref, batch_ref, wfc_ref, bfc_ref, out_ref,
                  s_acc, c_acc):
    i = pl.program_id(0)
    out2 = out2_ref[...]
    seg = (batch_ref[...] == lax.broadcasted_iota(jnp.int32, (RB, G), 1))
    seg = seg.astype(jnp.float32)
    part_s = lax.dot_general(seg, out2, (((0,), (0,)), ((), ())),
                             preferred_element_type=jnp.float32)
    ones = jnp.ones((RB, 1), jnp.float32)
    part_c = lax.dot_general(seg, ones, (((0,), (0,)), ((), ())),
                             preferred_element_type=jnp.float32)

    @pl.when(i == 0)
    def _():
        s_acc[...] = part_s
        c_acc[...] = part_c

    @pl.when(i > 0)
    def _():
        s_acc[...] += part_s
        c_acc[...] += part_c

    @pl.when(i == NBLK - 1)
    def _():
        pooled = s_acc[...] / jnp.maximum(c_acc[...], 1.0)
        out_ref[...] = (jnp.dot(pooled, wfc_ref[...],
                                preferred_element_type=jnp.float32)
                        + bfc_ref[...])


_tc_pool = pl.pallas_call(
    _tc_pool_body,
    grid=(NBLK,),
    in_specs=[
        pl.BlockSpec((RB, H), lambda i: (i, 0)),
        pl.BlockSpec((RB, 1), lambda i: (i, 0)),
        pl.BlockSpec((H, 1), lambda i: (0, 0)),
        pl.BlockSpec((1, 1), lambda i: (0, 0)),
    ],
    out_specs=pl.BlockSpec((G, 1), lambda i: (0, 0)),
    out_shape=jax.ShapeDtypeStruct((G, 1), jnp.float32),
    scratch_shapes=[
        pltpu.VMEM((G, H), jnp.float32),
        pltpu.VMEM((G, 1), jnp.float32),
    ],
)


def kernel(x, edge_index, batch, W1, b1, W2, b2, Wfc, bfc):
    pad = E_PAD - E
    srcp = jnp.concatenate([edge_index[0], jnp.zeros((pad,), jnp.int32)])
    dstp = jnp.concatenate([edge_index[1], jnp.full((pad,), N, jnp.int32)])

    degp = _sc_deg(dstp)
    g1, dinv = _tc_scale(x, W1, degp)

    agg1 = _sc_agg(g1, srcp, dstp)
    g2, _ = _tc_mid(agg1, g1, dinv, b1.reshape(1, H), W2)
    agg2 = _sc_agg(g2, srcp, dstp)
    _, out2 = _tc_mid(agg2, g2, dinv, b2.reshape(1, H), W2)

    out = _tc_pool(out2, batch.reshape(N, 1), Wfc, bfc.reshape(1, 1))
    return out.reshape(G)
